# Initial kernel scaffold; baseline (speedup 1.0000x reference)
import jax
import jax.numpy as jnp
from jax.experimental import pallas as pl


def kernel(pert_batch, W, b):
    raise NotImplementedError("write your pallas kernel here")


# speedup vs baseline: 19.2569x; 19.2569x over previous
import jax
import jax.numpy as jnp
from jax.experimental import pallas as pl


def _body(x_ref, w_ref, b_ref, o_ref):
    s = jnp.sum(x_ref[...], axis=1)
    y = jax.lax.dot_general(
        s, w_ref[...], (((1,), (1,)), ((), ())),
        preferred_element_type=jnp.float32,
    )
    o_ref[...] = y + b_ref[...]


def kernel(pert_batch, W, b):
    B, P, D = pert_batch.shape
    OUT = W.shape[0]
    BB = 512
    bias = (P * b).reshape(1, OUT)
    return pl.pallas_call(
        _body,
        grid=(B // BB,),
        in_specs=[
            pl.BlockSpec((BB, P, D), lambda i: (i, 0, 0)),
            pl.BlockSpec((OUT, D), lambda i: (0, 0)),
            pl.BlockSpec((1, OUT), lambda i: (0, 0)),
        ],
        out_specs=pl.BlockSpec((BB, OUT), lambda i: (i, 0)),
        out_shape=jax.ShapeDtypeStruct((B, OUT), jnp.float32),
    )(pert_batch, W, bias)
